# Initial kernel scaffold; baseline (speedup 1.0000x reference)
import jax
import jax.numpy as jnp
from jax.experimental import pallas as pl


def kernel(x, edge_index, edge_attr, p1ew, p1eb, p1prw, p1prb, p1psw, p1psb, p1lw, p1lb, p2ew, p2eb, p2prw, p2prb, p2psw, p2psb, p2lw, p2lb, gcw, gcb, f1w, f1b, f2w, f2b):
    raise NotImplementedError("write your pallas kernel here")


# speedup vs baseline: 4.6429x; 4.6429x over previous
import functools
import jax
import jax.numpy as jnp
from jax import lax
from jax.experimental import pallas as pl
from jax.experimental.pallas import tpu as pltpu
from jax.experimental.pallas import tpu_sc as plsc

N = 10000
E = 320000
NT = 32
NC = 2
NPW = 320
NPAD = NT * NPW
MAGIC = 52429
CHUNK = E // NT
EPAD = 327680

_mesh = functools.partial(
    plsc.VectorSubcoreMesh, core_axis_name="c", subcore_axis_name="s")
_SC_PARAMS = pltpu.CompilerParams(needs_layout_passes=False,
                                  use_tc_tiling_on_sc=False)


def _wid():
  return lax.axis_index("s") * NC + lax.axis_index("c")


def _iota16():
  return lax.iota(jnp.int32, 16)


def _sc_hist(dst):
  def body(dst_hbm, hist_hbm, dstw, hist, sem):
    w = _wid()
    pltpu.sync_copy(dst_hbm.at[pl.ds(w * CHUNK, CHUNK)], dstw)

    def zero_body(k, _):
      hist[pl.ds(k * 16, 16)] = jnp.zeros((16,), jnp.int32)
      return 0
    lax.fori_loop(0, NPAD // 16, zero_body, 0)

    ones = jnp.ones((16,), jnp.int32)

    def hist_body(k, _):
      v = dstw[pl.ds(k * 16, 16)]
      plsc.addupdate_scatter(hist, [v], ones)
      return 0
    lax.fori_loop(0, CHUNK // 16, hist_body, 0)
    pltpu.sync_copy(hist, hist_hbm.at[pl.ds(w * NPAD, NPAD)])
    del sem

  return pl.kernel(
      body,
      out_type=jax.ShapeDtypeStruct((NT * NPAD,), jnp.int32),
      mesh=_mesh(),
      compiler_params=_SC_PARAMS,
      scratch_types=[
          pltpu.VMEM((CHUNK,), jnp.int32),
          pltpu.VMEM((NPAD,), jnp.int32),
          pltpu.SemaphoreType.DMA,
      ],
  )(dst)


def _sc_offsets(hist):
  def body(hist_hbm, p_hbm, off_hbm, hcols, offcols, hrow, pv, tmp16):
    w = _wid()
    for t in range(NT):
      pltpu.sync_copy(hist_hbm.at[pl.ds(t * NPAD + w * NPW, NPW)],
                      hcols.at[pl.ds(t * NPW, NPW)])

    for kc in range(NPW // 16):
      run = jnp.zeros((16,), jnp.int32)
      for t in range(NT):
        v = hcols[pl.ds(t * NPW + kc * 16, 16)]
        offcols[pl.ds(t * NPW + kc * 16, 16)] = run
        run = run + v
      hrow[pl.ds(kc * 16, 16)] = run

    carry = jnp.zeros((), jnp.int32)
    for kc in range(NPW // 16):
      sl = pl.ds(kc * 16, 16)
      v = hrow[sl]
      c = plsc.cumsum(v)
      excl = c - v + jnp.full((16,), carry, jnp.int32)
      pv[sl] = excl
      carry = carry + lax.reduce_max(c, (0,))

    base = jnp.zeros((16,), jnp.int32)

    def row_base(t, b):
      pltpu.sync_copy(hist_hbm.at[pl.ds(t * NPAD, NPAD)], hrow)

      def acc(k, bb):
        return bb + hrow[pl.ds(k * 16, 16)]
      return lax.fori_loop(0, w * (NPW // 16), acc, b)
    base = lax.fori_loop(0, NT, row_base, base)
    base_sc = lax.reduce_sum(base, (0,))

    for kc in range(NPW // 16):
      sl = pl.ds(kc * 16, 16)
      pv[sl] = pv[sl] + jnp.full((16,), base_sc, jnp.int32)
    pltpu.sync_copy(pv, p_hbm.at[pl.ds(w * NPW, NPW)])

    @pl.when(w == NT - 1)
    def _():
      tmp16[...] = jnp.full((16,), E, jnp.int32)
      pltpu.sync_copy(tmp16, p_hbm.at[pl.ds(NPAD, 16)])

    for t in range(NT):
      pltpu.sync_copy(offcols.at[pl.ds(t * NPW, NPW)],
                      off_hbm.at[pl.ds(t * NPAD + w * NPW, NPW)])

  return pl.kernel(
      body,
      out_type=(
          jax.ShapeDtypeStruct((NPAD + 16,), jnp.int32),
          jax.ShapeDtypeStruct((NT * NPAD,), jnp.int32),
      ),
      mesh=_mesh(),
      compiler_params=_SC_PARAMS,
      scratch_types=[
          pltpu.VMEM((NT * NPW,), jnp.int32),
          pltpu.VMEM((NT * NPW,), jnp.int32),
          pltpu.VMEM((NPAD,), jnp.int32),
          pltpu.VMEM((NPW,), jnp.int32),
          pltpu.VMEM((16,), jnp.int32),
      ],
  )(hist)


def _sc_scatter(src, dst, ea_t_flat, p, off):
  def body(src_hbm, dst_hbm, ea_hbm, p_hbm, off_hbm,
           sp_hbm, e0_hbm, e1_hbm, e2_hbm, e3_hbm,
           srcw, dstw, eaw, myoff, pbuf, posb, rank_tmp, sem):
    w = _wid()
    pltpu.sync_copy(off_hbm.at[pl.ds(w * NPAD, NPAD)], myoff)
    pltpu.sync_copy(p_hbm.at[pl.ds(0, NPAD)], pbuf)

    def addp(k, _):
      sl = pl.ds(k * 16, 16)
      myoff[sl] = myoff[sl] + pbuf[sl]
      return 0
    lax.fori_loop(0, NPAD // 16, addp, 0)

    pltpu.sync_copy(src_hbm.at[pl.ds(w * CHUNK, CHUNK)], srcw)
    pltpu.sync_copy(dst_hbm.at[pl.ds(w * CHUNK, CHUNK)], dstw)
    for c in range(4):
      pltpu.sync_copy(ea_hbm.at[pl.ds(c * E + w * CHUNK, CHUNK)],
                      eaw.at[pl.ds(c * CHUNK, CHUNK)])

    iota = _iota16()
    ones = jnp.ones((16,), jnp.int32)

    def win(k, _):
      dv = dstw[pl.ds(k * 16, 16)]
      base = plsc.load_gather(myoff, [dv])
      sk, sv = plsc.sort_key_val(dv, iota)
      rank_tmp[...] = sk
      prev = plsc.load_gather(rank_tmp, [jnp.maximum(iota - 1, 0)])
      is_start = (sk != prev) | (iota == 0)
      runstart = plsc.cummax(jnp.where(is_start, iota, 0))
      rank_sorted = iota - runstart
      plsc.store_scatter(rank_tmp, [sv], rank_sorted)
      rank = rank_tmp[...]
      pos = base + rank
      posb[k // 5, pl.ds((k % 5) * 16, 16)] = pos
      plsc.addupdate_scatter(myoff, [dv], ones)
      return 0
    lax.fori_loop(0, CHUNK // 16, win, 0)

    nrow = CHUNK // 80
    wave = 5
    ea_outs = (e0_hbm, e1_hbm, e2_hbm, e3_hbm)
    for r0 in range(0, nrow, wave):
      cps = []
      for r in range(r0, r0 + wave):
        cps.append(pltpu.async_copy(
            srcw.at[pl.ds(r * 80, 80)], sp_hbm.at[posb.at[r]], sem))
        for c in range(4):
          cps.append(pltpu.async_copy(
              eaw.at[pl.ds(c * CHUNK + r * 80, 80)],
              ea_outs[c].at[posb.at[r]], sem))
      for cp in cps:
        cp.wait()

  return pl.kernel(
      body,
      out_type=(
          jax.ShapeDtypeStruct((EPAD,), jnp.int32),
          jax.ShapeDtypeStruct((EPAD,), jnp.float32),
          jax.ShapeDtypeStruct((EPAD,), jnp.float32),
          jax.ShapeDtypeStruct((EPAD,), jnp.float32),
          jax.ShapeDtypeStruct((EPAD,), jnp.float32),
      ),
      mesh=_mesh(),
      compiler_params=_SC_PARAMS,
      scratch_types=[
          pltpu.VMEM((CHUNK,), jnp.int32),
          pltpu.VMEM((CHUNK,), jnp.int32),
          pltpu.VMEM((4 * CHUNK,), jnp.float32),
          pltpu.VMEM((NPAD,), jnp.int32),
          pltpu.VMEM((NPAD,), jnp.int32),
          pltpu.VMEM((CHUNK // 80, 80), jnp.int32),
          pltpu.VMEM((16,), jnp.int32),
          pltpu.SemaphoreType.DMA,
      ],
  )(src, dst, ea_t_flat, p, off)


def _sc_accumulate(srcp2d, c_rows, b_tab, p, F, KW, with_c):
  NQ = F // 16
  NS = KW // 128

  def body(*refs):
    if with_c:
      (srcp_hbm, c_hbm, b_hbm, p_hbm,
       s1_hbm, s2_hbm, mx_hbm, mn_hbm,
       idxv, brows, crows, s1l, s2l, mxl, mnl,
       psp, psm, wsm, sem) = refs
      outs_l = (s1l, s2l, mxl, mnl)
      outs_h = (s1_hbm, s2_hbm, mx_hbm, mn_hbm)
    else:
      (srcp_hbm, b_hbm, p_hbm, s1_hbm,
       idxv, brows, s1l, psp, psm, wsm, sem) = refs
      c_hbm = None
      crows = None
      outs_l = (s1l,)
      outs_h = (s1_hbm,)

    w = _wid()
    pltpu.sync_copy(p_hbm.at[pl.ds(w * NPW, NPW + 8)],
                    psp.at[pl.ds(w * 328, 328)])
    pltpu.sync_copy(psp.at[pl.ds(w * 328, 328)], psm)

    def load_window(wstart):
      pltpu.sync_copy(srcp_hbm.at[pl.ds(wstart, KW)], idxv)
      for j in range(KW // 16):
        sl = pl.ds(j * 16, 16)
        v = idxv[sl]
        idxv[sl] = jnp.minimum(
            jnp.maximum(v, jnp.zeros((16,), jnp.int32)),
            jnp.full((16,), NPAD - 1, jnp.int32))
      cps = []
      for j in range(NS):
        cps.append(pltpu.async_copy(
            b_hbm.at[idxv.at[pl.ds(j * 128, 128)]],
            brows.at[pl.ds(j * 128, 128)], sem))
      for cp in cps:
        cp.wait()
      if with_c:
        pltpu.sync_copy(c_hbm.at[pl.ds(wstart, KW)], crows)

    e0 = psm[0]
    w0 = (e0 // 128) * 128
    wsm[0] = w0
    load_window(w0)

    neg = jnp.full((16,), -3.0e38, jnp.float32)
    pos = jnp.full((16,), 3.0e38, jnp.float32)
    zero = jnp.zeros((16,), jnp.float32)

    def body_d(i, _):
      s = psm[i]
      e1 = psm[i + 1]
      if with_c:
        acc0 = ([zero] * NQ, [zero] * NQ, [neg] * NQ, [pos] * NQ)
      else:
        acc0 = ([zero] * NQ,)

      def body_e(e, accs):
        ws = wsm[0]

        @pl.when(e - ws >= KW)
        def _():
          nw = (e // 128) * 128
          wsm[0] = nw
          load_window(nw)

        off = e - wsm[0]
        if with_c:
          s1a, s2a, mxa, mna = accs
          ns1, ns2, nmx, nmn = [], [], [], []
          for q in range(NQ):
            sl = pl.ds(q * 16, 16)
            t = brows[off, sl] + crows[off, sl]
            ns1.append(s1a[q] + t)
            ns2.append(s2a[q] + t * t)
            nmx.append(jnp.maximum(mxa[q], t))
            nmn.append(jnp.minimum(mna[q], t))
          return (ns1, ns2, nmx, nmn)
        else:
          (s1a,) = accs
          ns1 = []
          for q in range(NQ):
            sl = pl.ds(q * 16, 16)
            ns1.append(s1a[q] + brows[off, sl])
          return (ns1,)

      accs = lax.fori_loop(s, e1, body_e, acc0)
      for ai, al in enumerate(outs_l):
        for q in range(NQ):
          al[i, pl.ds(q * 16, 16)] = accs[ai][q]
      return 0

    lax.fori_loop(0, NPW, body_d, 0)
    for al, ah in zip(outs_l, outs_h):
      pltpu.sync_copy(al, ah.at[pl.ds(w * NPW, NPW)])

  n_out = 4 if with_c else 1
  out_type = tuple(
      jax.ShapeDtypeStruct((NPAD, F), jnp.float32) for _ in range(n_out))
  if n_out == 1:
    out_type = out_type[0]
  scratch = [pltpu.VMEM((KW,), jnp.int32),
             pltpu.VMEM((KW, F), jnp.float32)]
  if with_c:
    scratch.append(pltpu.VMEM((KW, F), jnp.float32))
  scratch += [pltpu.VMEM((NPW, F), jnp.float32) for _ in range(n_out)]
  scratch += [
      pltpu.VMEM_SHARED((NT * 328,), jnp.int32),
      pltpu.SMEM((NPW + 8,), jnp.int32),
      pltpu.SMEM((8,), jnp.int32),
      pltpu.SemaphoreType.DMA,
  ]
  args = (srcp2d, c_rows, b_tab, p) if with_c else (srcp2d, b_tab, p)
  return pl.kernel(
      body, out_type=out_type, mesh=_mesh(),
      compiler_params=_SC_PARAMS, scratch_types=scratch)(*args)


def _dot(a, b):
  return jnp.dot(a, b, preferred_element_type=jnp.float32)


BLKN = 2048


def _tc_node_pre(xp, wi, wj):
  def body(x_ref, wi_ref, wj_ref, a_ref, b_ref):
    x = x_ref[...]
    a_ref[...] = _dot(x, wi_ref[...])
    b_ref[...] = _dot(x, wj_ref[...])
  return pl.pallas_call(
      body,
      grid=(NPAD // BLKN,),
      in_specs=[
          pl.BlockSpec((BLKN, 16), lambda i: (i, 0)),
          pl.BlockSpec((16, 16), lambda i: (0, 0)),
          pl.BlockSpec((16, 16), lambda i: (0, 0)),
      ],
      out_specs=(pl.BlockSpec((BLKN, 16), lambda i: (i, 0)),
                 pl.BlockSpec((BLKN, 16), lambda i: (i, 0))),
      out_shape=(jax.ShapeDtypeStruct((NPAD, 16), jnp.float32),
                 jax.ShapeDtypeStruct((NPAD, 16), jnp.float32)),
  )(xp, wi, wj)


def _tc_edge_c(eap, ew1, eb1, we1, prb1, ew2, eb2, we2, prb2):
  BLK = 2048

  def body(ea_ref, ew1_ref, eb1_ref, we1_ref, prb1_ref,
           ew2_ref, eb2_ref, we2_ref, prb2_ref, c1_ref, c2_ref):
    ea = ea_ref[...]
    ea1 = _dot(ea, ew1_ref[...]) + eb1_ref[...]
    c1_ref[...] = _dot(ea1, we1_ref[...]) + prb1_ref[...]
    ea2 = _dot(ea, ew2_ref[...]) + eb2_ref[...]
    c2_ref[...] = _dot(ea2, we2_ref[...]) + prb2_ref[...]

  grid = EPAD // BLK
  return pl.pallas_call(
      body,
      grid=(grid,),
      in_specs=[
          pl.BlockSpec((BLK, 4), lambda i: (i, 0)),
          pl.BlockSpec((4, 16), lambda i: (0, 0)),
          pl.BlockSpec((1, 16), lambda i: (0, 0)),
          pl.BlockSpec((16, 16), lambda i: (0, 0)),
          pl.BlockSpec((1, 16), lambda i: (0, 0)),
          pl.BlockSpec((4, 64), lambda i: (0, 0)),
          pl.BlockSpec((1, 64), lambda i: (0, 0)),
          pl.BlockSpec((64, 64), lambda i: (0, 0)),
          pl.BlockSpec((1, 64), lambda i: (0, 0)),
      ],
      out_specs=(
          pl.BlockSpec((BLK, 16), lambda i: (i, 0)),
          pl.BlockSpec((BLK, 64), lambda i: (i, 0)),
      ),
      out_shape=(jax.ShapeDtypeStruct((EPAD, 16), jnp.float32),
                 jax.ShapeDtypeStruct((EPAD, 64), jnp.float32)),
  )(eap, ew1, eb1, we1, prb1, ew2, eb2, we2, prb2)


def _pna_combine(x, a, s1, s2, mx, mn, cnt, psw, psb, lw, lb):
  cntc = jnp.maximum(cnt, 1.0)
  s = cnt * a + s1
  mean = s / cntc
  pos_deg = cnt > 0.0
  mxo = jnp.where(pos_deg, a + mx, 0.0)
  mno = jnp.where(pos_deg, a + mn, 0.0)
  m1 = s1 / cntc
  var = s2 / cntc - m1 * m1
  std = jnp.sqrt(jax.nn.relu(var) + 1e-5)
  fi = x.shape[1]
  out = (_dot(x, psw[0:fi]) + _dot(s, psw[fi:2 * fi])
         + _dot(mean, psw[2 * fi:3 * fi]) + _dot(mxo, psw[3 * fi:4 * fi])
         + _dot(mno, psw[4 * fi:5 * fi]) + _dot(std, psw[5 * fi:6 * fi])
         + psb)
  return _dot(out, lw) + lb


def _tc_combine1(xp, a1, s1, s2, mx, mn, cnt, psw, psb, lw, lb, wi2, wj2):
  def body(x_ref, a_ref, s1_ref, s2_ref, mx_ref, mn_ref, cnt_ref,
           psw_ref, psb_ref, lw_ref, lb_ref, wi2_ref, wj2_ref,
           h1_ref, a2_ref, b2_ref):
    out1 = _pna_combine(x_ref[...], a_ref[...], s1_ref[...], s2_ref[...],
                        mx_ref[...], mn_ref[...], cnt_ref[...],
                        psw_ref[...], psb_ref[...], lw_ref[...], lb_ref[...])
    h1 = jax.nn.relu(out1)
    h1_ref[...] = h1
    a2_ref[...] = _dot(h1, wi2_ref[...])
    b2_ref[...] = _dot(h1, wj2_ref[...])
  def bs(r, c):
    return pl.BlockSpec((r, c), lambda i: (i, 0))

  def ws(r, c):
    return pl.BlockSpec((r, c), lambda i: (0, 0))

  return pl.pallas_call(
      body,
      grid=(NPAD // BLKN,),
      in_specs=[bs(BLKN, 16), bs(BLKN, 16), bs(BLKN, 16), bs(BLKN, 16),
                bs(BLKN, 16), bs(BLKN, 16), bs(BLKN, 1),
                ws(96, 64), ws(1, 64), ws(64, 64), ws(1, 64),
                ws(64, 64), ws(64, 64)],
      out_specs=(bs(BLKN, 64), bs(BLKN, 64), bs(BLKN, 64)),
      out_shape=(jax.ShapeDtypeStruct((NPAD, 64), jnp.float32),
                 jax.ShapeDtypeStruct((NPAD, 64), jnp.float32),
                 jax.ShapeDtypeStruct((NPAD, 64), jnp.float32)),
  )(xp, a1, s1, s2, mx, mn, cnt, psw, psb, lw, lb, wi2, wj2)


def _tc_combine2(h1, a2, s1, s2, mx, mn, cnt, psw, psb, lw, lb, gcw):
  def body(h_ref, a_ref, s1_ref, s2_ref, mx_ref, mn_ref, cnt_ref,
           psw_ref, psb_ref, lw_ref, lb_ref, gcw_ref, u_ref, dinv_ref):
    out2 = _pna_combine(h_ref[...], a_ref[...], s1_ref[...], s2_ref[...],
                        mx_ref[...], mn_ref[...], cnt_ref[...],
                        psw_ref[...], psb_ref[...], lw_ref[...], lb_ref[...])
    out2 = jax.nn.relu(out2)
    xw = _dot(out2, gcw_ref[...])
    dinv = lax.rsqrt(cnt_ref[...] + 1.0)
    dinv_ref[...] = dinv
    u_ref[...] = xw * dinv
  def bs(r, c):
    return pl.BlockSpec((r, c), lambda i: (i, 0))

  def ws(r, c):
    return pl.BlockSpec((r, c), lambda i: (0, 0))

  return pl.pallas_call(
      body,
      grid=(NPAD // BLKN,),
      in_specs=[bs(BLKN, 64), bs(BLKN, 64), bs(BLKN, 64), bs(BLKN, 64),
                bs(BLKN, 64), bs(BLKN, 64), bs(BLKN, 1),
                ws(384, 64), ws(1, 64), ws(64, 64), ws(1, 64), ws(64, 16)],
      out_specs=(bs(BLKN, 16), bs(BLKN, 1)),
      out_shape=(jax.ShapeDtypeStruct((NPAD, 16), jnp.float32),
                 jax.ShapeDtypeStruct((NPAD, 1), jnp.float32)),
  )(h1, a2, s1, s2, mx, mn, cnt, psw, psb, lw, lb, gcw)


def _tc_final(sg, u, dinv, xp, gcb, f1w, f1b, f2w, f2b):
  def body(sg_ref, u_ref, dinv_ref, x_ref, gcb_ref,
           f1w_ref, f1b_ref, f2w_ref, f2b_ref, o_ref):
    gout = dinv_ref[...] * (sg_ref[...] + u_ref[...]) + gcb_ref[...]
    h = jax.nn.relu(_dot(gout, f1w_ref[0:16]) + _dot(x_ref[...], f1w_ref[16:32])
                    + f1b_ref[...])
    o_ref[...] = _dot(h, f2w_ref[...]) + f2b_ref[...]
  def bs(r, c):
    return pl.BlockSpec((r, c), lambda i: (i, 0))

  def ws(r, c):
    return pl.BlockSpec((r, c), lambda i: (0, 0))

  return pl.pallas_call(
      body,
      grid=(NPAD // BLKN,),
      in_specs=[bs(BLKN, 16), bs(BLKN, 16), bs(BLKN, 1), bs(BLKN, 16),
                ws(1, 16), ws(32, 10), ws(1, 10), ws(10, 10), ws(1, 10)],
      out_specs=bs(BLKN, 10),
      out_shape=jax.ShapeDtypeStruct((NPAD, 10), jnp.float32),
  )(sg, u, dinv, xp, gcb, f1w, f1b, f2w, f2b)


def kernel(x, edge_index, edge_attr, p1ew, p1eb, p1prw, p1prb, p1psw, p1psb,
           p1lw, p1lb, p2ew, p2eb, p2prw, p2prb, p2psw, p2psb, p2lw, p2lb,
           gcw, gcb, f1w, f1b, f2w, f2b):
  src = edge_index[0]
  dst = edge_index[1]
  xp = jnp.pad(x, ((0, NPAD - N), (0, 0)))

  wi1, wj1, we1 = p1prw[0:16], p1prw[16:32], p1prw[32:48]
  wi2, wj2, we2 = p2prw[0:64], p2prw[64:128], p2prw[128:192]

  hist = _sc_hist(dst)
  p, off = _sc_offsets(hist)
  ea_t_flat = edge_attr.T.reshape(-1)
  src_perm, eac0, eac1, eac2, eac3 = _sc_scatter(src, dst, ea_t_flat, p, off)
  ea_perm = jnp.stack([eac0, eac1, eac2, eac3], axis=1)
  srcp2d = src_perm

  cnt = (p[1:NPAD + 1] - p[0:NPAD]).astype(jnp.float32)[:, None]

  a1, b1 = _tc_node_pre(xp, wi1, wj1)
  c1, c2 = _tc_edge_c(ea_perm, p1ew, p1eb[None, :], we1, p1prb[None, :],
                      p2ew, p2eb[None, :], we2, p2prb[None, :])

  s1, s2, mx, mn = _sc_accumulate(srcp2d, c1, b1, p, 16, 1024, True)
  h1, a2, b2 = _tc_combine1(xp, a1, s1, s2, mx, mn, cnt,
                            p1psw, p1psb[None, :], p1lw, p1lb[None, :],
                            wi2, wj2)

  s1b, s2b, mxb, mnb = _sc_accumulate(srcp2d, c2, b2, p, 64, 256, True)
  u, dinv = _tc_combine2(h1, a2, s1b, s2b, mxb, mnb, cnt,
                         p2psw, p2psb[None, :], p2lw, p2lb[None, :], gcw)

  sg = _sc_accumulate(srcp2d, None, u, p, 16, 1024, False)
  out = _tc_final(sg, u, dinv, xp, gcb[None, :], f1w, f1b[None, :],
                  f2w, f2b[None, :])
  return out[:N]


# speedup vs baseline: 4.6879x; 1.0097x over previous
import functools
import jax
import jax.numpy as jnp
from jax import lax
from jax.experimental import pallas as pl
from jax.experimental.pallas import tpu as pltpu
from jax.experimental.pallas import tpu_sc as plsc

N = 10000
E = 320000
NT = 32
NC = 2
NPW = 320
NPAD = NT * NPW
MAGIC = 52429
CHUNK = E // NT
EPAD = 327680

_mesh = functools.partial(
    plsc.VectorSubcoreMesh, core_axis_name="c", subcore_axis_name="s")
_SC_PARAMS = pltpu.CompilerParams(needs_layout_passes=False,
                                  use_tc_tiling_on_sc=False)


def _wid():
  return lax.axis_index("s") * NC + lax.axis_index("c")


def _iota16():
  return lax.iota(jnp.int32, 16)


def _sc_hist(dst):
  def body(dst_hbm, hist_hbm, dstw, hist, sem):
    w = _wid()
    pltpu.sync_copy(dst_hbm.at[pl.ds(w * CHUNK, CHUNK)], dstw)

    def zero_body(k, _):
      hist[pl.ds(k * 16, 16)] = jnp.zeros((16,), jnp.int32)
      return 0
    lax.fori_loop(0, NPAD // 16, zero_body, 0)

    ones = jnp.ones((16,), jnp.int32)

    def hist_body(k, _):
      v = dstw[pl.ds(k * 16, 16)]
      plsc.addupdate_scatter(hist, [v], ones)
      return 0
    lax.fori_loop(0, CHUNK // 16, hist_body, 0)
    pltpu.sync_copy(hist, hist_hbm.at[pl.ds(w * NPAD, NPAD)])
    del sem

  return pl.kernel(
      body,
      out_type=jax.ShapeDtypeStruct((NT * NPAD,), jnp.int32),
      mesh=_mesh(),
      compiler_params=_SC_PARAMS,
      scratch_types=[
          pltpu.VMEM((CHUNK,), jnp.int32),
          pltpu.VMEM((NPAD,), jnp.int32),
          pltpu.SemaphoreType.DMA,
      ],
  )(dst)


def _sc_offsets(hist):
  def body(hist_hbm, p_hbm, off_hbm, hcols, offcols, hrow, pv, tmp16):
    w = _wid()
    for t in range(NT):
      pltpu.sync_copy(hist_hbm.at[pl.ds(t * NPAD + w * NPW, NPW)],
                      hcols.at[pl.ds(t * NPW, NPW)])

    for kc in range(NPW // 16):
      run = jnp.zeros((16,), jnp.int32)
      for t in range(NT):
        v = hcols[pl.ds(t * NPW + kc * 16, 16)]
        offcols[pl.ds(t * NPW + kc * 16, 16)] = run
        run = run + v
      hrow[pl.ds(kc * 16, 16)] = run

    carry = jnp.zeros((), jnp.int32)
    for kc in range(NPW // 16):
      sl = pl.ds(kc * 16, 16)
      v = hrow[sl]
      c = plsc.cumsum(v)
      excl = c - v + jnp.full((16,), carry, jnp.int32)
      pv[sl] = excl
      carry = carry + lax.reduce_max(c, (0,))

    base = jnp.zeros((16,), jnp.int32)

    def row_base(t, b):
      pltpu.sync_copy(hist_hbm.at[pl.ds(t * NPAD, NPAD)], hrow)

      def acc(k, bb):
        return bb + hrow[pl.ds(k * 16, 16)]
      return lax.fori_loop(0, w * (NPW // 16), acc, b)
    base = lax.fori_loop(0, NT, row_base, base)
    base_sc = lax.reduce_sum(base, (0,))

    for kc in range(NPW // 16):
      sl = pl.ds(kc * 16, 16)
      pv[sl] = pv[sl] + jnp.full((16,), base_sc, jnp.int32)
    pltpu.sync_copy(pv, p_hbm.at[pl.ds(w * NPW, NPW)])

    @pl.when(w == NT - 1)
    def _():
      tmp16[...] = jnp.full((16,), E, jnp.int32)
      pltpu.sync_copy(tmp16, p_hbm.at[pl.ds(NPAD, 16)])

    for t in range(NT):
      pltpu.sync_copy(offcols.at[pl.ds(t * NPW, NPW)],
                      off_hbm.at[pl.ds(t * NPAD + w * NPW, NPW)])

  return pl.kernel(
      body,
      out_type=(
          jax.ShapeDtypeStruct((NPAD + 16,), jnp.int32),
          jax.ShapeDtypeStruct((NT * NPAD,), jnp.int32),
      ),
      mesh=_mesh(),
      compiler_params=_SC_PARAMS,
      scratch_types=[
          pltpu.VMEM((NT * NPW,), jnp.int32),
          pltpu.VMEM((NT * NPW,), jnp.int32),
          pltpu.VMEM((NPAD,), jnp.int32),
          pltpu.VMEM((NPW,), jnp.int32),
          pltpu.VMEM((16,), jnp.int32),
      ],
  )(hist)


def _sc_scatter(src, dst, ea_t_flat, p, off):
  def body(src_hbm, dst_hbm, ea_hbm, p_hbm, off_hbm,
           sp_hbm, e0_hbm, e1_hbm, e2_hbm, e3_hbm,
           srcw, dstw, eaw, myoff, pbuf, posb, rank_tmp, sem):
    w = _wid()
    pltpu.sync_copy(off_hbm.at[pl.ds(w * NPAD, NPAD)], myoff)
    pltpu.sync_copy(p_hbm.at[pl.ds(0, NPAD)], pbuf)

    def addp(k, _):
      sl = pl.ds(k * 16, 16)
      myoff[sl] = myoff[sl] + pbuf[sl]
      return 0
    lax.fori_loop(0, NPAD // 16, addp, 0)

    pltpu.sync_copy(src_hbm.at[pl.ds(w * CHUNK, CHUNK)], srcw)
    pltpu.sync_copy(dst_hbm.at[pl.ds(w * CHUNK, CHUNK)], dstw)
    for c in range(4):
      pltpu.sync_copy(ea_hbm.at[pl.ds(c * E + w * CHUNK, CHUNK)],
                      eaw.at[pl.ds(c * CHUNK, CHUNK)])

    iota = _iota16()
    ones = jnp.ones((16,), jnp.int32)

    def win(k, _):
      dv = dstw[pl.ds(k * 16, 16)]
      base = plsc.load_gather(myoff, [dv])
      sk, sv = plsc.sort_key_val(dv, iota)
      rank_tmp[...] = sk
      prev = plsc.load_gather(rank_tmp, [jnp.maximum(iota - 1, 0)])
      is_start = (sk != prev) | (iota == 0)
      runstart = plsc.cummax(jnp.where(is_start, iota, 0))
      rank_sorted = iota - runstart
      plsc.store_scatter(rank_tmp, [sv], rank_sorted)
      rank = rank_tmp[...]
      pos = base + rank
      posb[pl.ds(k * 16, 16)] = pos
      plsc.addupdate_scatter(myoff, [dv], ones)
      return 0
    lax.fori_loop(0, CHUNK // 16, win, 0)

    ea_outs = (e0_hbm, e1_hbm, e2_hbm, e3_hbm)
    cps = [pltpu.async_copy(srcw, sp_hbm.at[posb], sem)]
    for c in range(4):
      cps.append(pltpu.async_copy(
          eaw.at[pl.ds(c * CHUNK, CHUNK)], ea_outs[c].at[posb], sem))
    for cp in cps:
      cp.wait()

  return pl.kernel(
      body,
      out_type=(
          jax.ShapeDtypeStruct((EPAD,), jnp.int32),
          jax.ShapeDtypeStruct((EPAD,), jnp.float32),
          jax.ShapeDtypeStruct((EPAD,), jnp.float32),
          jax.ShapeDtypeStruct((EPAD,), jnp.float32),
          jax.ShapeDtypeStruct((EPAD,), jnp.float32),
      ),
      mesh=_mesh(),
      compiler_params=_SC_PARAMS,
      scratch_types=[
          pltpu.VMEM((CHUNK,), jnp.int32),
          pltpu.VMEM((CHUNK,), jnp.int32),
          pltpu.VMEM((4 * CHUNK,), jnp.float32),
          pltpu.VMEM((NPAD,), jnp.int32),
          pltpu.VMEM((NPAD,), jnp.int32),
          pltpu.VMEM((CHUNK,), jnp.int32),
          pltpu.VMEM((16,), jnp.int32),
          pltpu.SemaphoreType.DMA,
      ],
  )(src, dst, ea_t_flat, p, off)


def _sc_accumulate(srcp2d, c_rows, b_tab, p, F, KW, with_c):
  NQ = F // 16
  NS = KW // 128

  def body(*refs):
    if with_c:
      (srcp_hbm, c_hbm, b_hbm, p_hbm,
       s1_hbm, s2_hbm, mx_hbm, mn_hbm,
       idxv, brows, crows, s1l, s2l, mxl, mnl,
       psp, psm, wsm, sem) = refs
      outs_l = (s1l, s2l, mxl, mnl)
      outs_h = (s1_hbm, s2_hbm, mx_hbm, mn_hbm)
    else:
      (srcp_hbm, b_hbm, p_hbm, s1_hbm,
       idxv, brows, s1l, psp, psm, wsm, sem) = refs
      c_hbm = None
      crows = None
      outs_l = (s1l,)
      outs_h = (s1_hbm,)

    w = _wid()
    pltpu.sync_copy(p_hbm.at[pl.ds(w * NPW, NPW + 8)],
                    psp.at[pl.ds(w * 328, 328)])
    pltpu.sync_copy(psp.at[pl.ds(w * 328, 328)], psm)

    def load_window(wstart):
      pltpu.sync_copy(srcp_hbm.at[pl.ds(wstart, KW)], idxv)
      for j in range(KW // 16):
        sl = pl.ds(j * 16, 16)
        v = idxv[sl]
        idxv[sl] = jnp.minimum(
            jnp.maximum(v, jnp.zeros((16,), jnp.int32)),
            jnp.full((16,), NPAD - 1, jnp.int32))
      cp = pltpu.async_copy(b_hbm.at[idxv], brows, sem)
      if with_c:
        pltpu.sync_copy(c_hbm.at[pl.ds(wstart, KW)], crows)
      cp.wait()

    e0 = psm[0]
    w0 = (e0 // 128) * 128
    wsm[0] = w0
    load_window(w0)

    neg = jnp.full((16,), -3.0e38, jnp.float32)
    pos = jnp.full((16,), 3.0e38, jnp.float32)
    zero = jnp.zeros((16,), jnp.float32)

    def body_d(i, _):
      s = psm[i]
      e1 = psm[i + 1]
      if with_c:
        acc0 = ([zero] * NQ, [zero] * NQ, [neg] * NQ, [pos] * NQ)
      else:
        acc0 = ([zero] * NQ,)

      def body_e(e, accs):
        ws = wsm[0]

        @pl.when(e - ws >= KW)
        def _():
          nw = (e // 128) * 128
          wsm[0] = nw
          load_window(nw)

        off = e - wsm[0]
        if with_c:
          s1a, s2a, mxa, mna = accs
          ns1, ns2, nmx, nmn = [], [], [], []
          for q in range(NQ):
            sl = pl.ds(q * 16, 16)
            t = brows[off, sl] + crows[off, sl]
            ns1.append(s1a[q] + t)
            ns2.append(s2a[q] + t * t)
            nmx.append(jnp.maximum(mxa[q], t))
            nmn.append(jnp.minimum(mna[q], t))
          return (ns1, ns2, nmx, nmn)
        else:
          (s1a,) = accs
          ns1 = []
          for q in range(NQ):
            sl = pl.ds(q * 16, 16)
            ns1.append(s1a[q] + brows[off, sl])
          return (ns1,)

      accs = lax.fori_loop(s, e1, body_e, acc0)
      for ai, al in enumerate(outs_l):
        for q in range(NQ):
          al[i, pl.ds(q * 16, 16)] = accs[ai][q]
      return 0

    lax.fori_loop(0, NPW, body_d, 0)
    for al, ah in zip(outs_l, outs_h):
      pltpu.sync_copy(al, ah.at[pl.ds(w * NPW, NPW)])

  n_out = 4 if with_c else 1
  out_type = tuple(
      jax.ShapeDtypeStruct((NPAD, F), jnp.float32) for _ in range(n_out))
  if n_out == 1:
    out_type = out_type[0]
  scratch = [pltpu.VMEM((KW,), jnp.int32),
             pltpu.VMEM((KW, F), jnp.float32)]
  if with_c:
    scratch.append(pltpu.VMEM((KW, F), jnp.float32))
  scratch += [pltpu.VMEM((NPW, F), jnp.float32) for _ in range(n_out)]
  scratch += [
      pltpu.VMEM_SHARED((NT * 328,), jnp.int32),
      pltpu.SMEM((NPW + 8,), jnp.int32),
      pltpu.SMEM((8,), jnp.int32),
      pltpu.SemaphoreType.DMA,
  ]
  args = (srcp2d, c_rows, b_tab, p) if with_c else (srcp2d, b_tab, p)
  return pl.kernel(
      body, out_type=out_type, mesh=_mesh(),
      compiler_params=_SC_PARAMS, scratch_types=scratch)(*args)


def _dot(a, b):
  return jnp.dot(a, b, preferred_element_type=jnp.float32)


BLKN = 2048


def _tc_node_pre(xp, wi, wj):
  def body(x_ref, wi_ref, wj_ref, a_ref, b_ref):
    x = x_ref[...]
    a_ref[...] = _dot(x, wi_ref[...])
    b_ref[...] = _dot(x, wj_ref[...])
  return pl.pallas_call(
      body,
      grid=(NPAD // BLKN,),
      in_specs=[
          pl.BlockSpec((BLKN, 16), lambda i: (i, 0)),
          pl.BlockSpec((16, 16), lambda i: (0, 0)),
          pl.BlockSpec((16, 16), lambda i: (0, 0)),
      ],
      out_specs=(pl.BlockSpec((BLKN, 16), lambda i: (i, 0)),
                 pl.BlockSpec((BLKN, 16), lambda i: (i, 0))),
      out_shape=(jax.ShapeDtypeStruct((NPAD, 16), jnp.float32),
                 jax.ShapeDtypeStruct((NPAD, 16), jnp.float32)),
  )(xp, wi, wj)


def _tc_edge_c(eap, ew1, eb1, we1, prb1, ew2, eb2, we2, prb2):
  BLK = 2048

  def body(ea_ref, ew1_ref, eb1_ref, we1_ref, prb1_ref,
           ew2_ref, eb2_ref, we2_ref, prb2_ref, c1_ref, c2_ref):
    ea = ea_ref[...]
    ea1 = _dot(ea, ew1_ref[...]) + eb1_ref[...]
    c1_ref[...] = _dot(ea1, we1_ref[...]) + prb1_ref[...]
    ea2 = _dot(ea, ew2_ref[...]) + eb2_ref[...]
    c2_ref[...] = _dot(ea2, we2_ref[...]) + prb2_ref[...]

  grid = EPAD // BLK
  return pl.pallas_call(
      body,
      grid=(grid,),
      in_specs=[
          pl.BlockSpec((BLK, 4), lambda i: (i, 0)),
          pl.BlockSpec((4, 16), lambda i: (0, 0)),
          pl.BlockSpec((1, 16), lambda i: (0, 0)),
          pl.BlockSpec((16, 16), lambda i: (0, 0)),
          pl.BlockSpec((1, 16), lambda i: (0, 0)),
          pl.BlockSpec((4, 64), lambda i: (0, 0)),
          pl.BlockSpec((1, 64), lambda i: (0, 0)),
          pl.BlockSpec((64, 64), lambda i: (0, 0)),
          pl.BlockSpec((1, 64), lambda i: (0, 0)),
      ],
      out_specs=(
          pl.BlockSpec((BLK, 16), lambda i: (i, 0)),
          pl.BlockSpec((BLK, 64), lambda i: (i, 0)),
      ),
      out_shape=(jax.ShapeDtypeStruct((EPAD, 16), jnp.float32),
                 jax.ShapeDtypeStruct((EPAD, 64), jnp.float32)),
  )(eap, ew1, eb1, we1, prb1, ew2, eb2, we2, prb2)


def _pna_combine(x, a, s1, s2, mx, mn, cnt, psw, psb, lw, lb):
  cntc = jnp.maximum(cnt, 1.0)
  s = cnt * a + s1
  mean = s / cntc
  pos_deg = cnt > 0.0
  mxo = jnp.where(pos_deg, a + mx, 0.0)
  mno = jnp.where(pos_deg, a + mn, 0.0)
  m1 = s1 / cntc
  var = s2 / cntc - m1 * m1
  std = jnp.sqrt(jax.nn.relu(var) + 1e-5)
  fi = x.shape[1]
  out = (_dot(x, psw[0:fi]) + _dot(s, psw[fi:2 * fi])
         + _dot(mean, psw[2 * fi:3 * fi]) + _dot(mxo, psw[3 * fi:4 * fi])
         + _dot(mno, psw[4 * fi:5 * fi]) + _dot(std, psw[5 * fi:6 * fi])
         + psb)
  return _dot(out, lw) + lb


def _tc_combine1(xp, a1, s1, s2, mx, mn, cnt, psw, psb, lw, lb, wi2, wj2):
  def body(x_ref, a_ref, s1_ref, s2_ref, mx_ref, mn_ref, cnt_ref,
           psw_ref, psb_ref, lw_ref, lb_ref, wi2_ref, wj2_ref,
           h1_ref, a2_ref, b2_ref):
    out1 = _pna_combine(x_ref[...], a_ref[...], s1_ref[...], s2_ref[...],
                        mx_ref[...], mn_ref[...], cnt_ref[...],
                        psw_ref[...], psb_ref[...], lw_ref[...], lb_ref[...])
    h1 = jax.nn.relu(out1)
    h1_ref[...] = h1
    a2_ref[...] = _dot(h1, wi2_ref[...])
    b2_ref[...] = _dot(h1, wj2_ref[...])
  def bs(r, c):
    return pl.BlockSpec((r, c), lambda i: (i, 0))

  def ws(r, c):
    return pl.BlockSpec((r, c), lambda i: (0, 0))

  return pl.pallas_call(
      body,
      grid=(NPAD // BLKN,),
      in_specs=[bs(BLKN, 16), bs(BLKN, 16), bs(BLKN, 16), bs(BLKN, 16),
                bs(BLKN, 16), bs(BLKN, 16), bs(BLKN, 1),
                ws(96, 64), ws(1, 64), ws(64, 64), ws(1, 64),
                ws(64, 64), ws(64, 64)],
      out_specs=(bs(BLKN, 64), bs(BLKN, 64), bs(BLKN, 64)),
      out_shape=(jax.ShapeDtypeStruct((NPAD, 64), jnp.float32),
                 jax.ShapeDtypeStruct((NPAD, 64), jnp.float32),
                 jax.ShapeDtypeStruct((NPAD, 64), jnp.float32)),
  )(xp, a1, s1, s2, mx, mn, cnt, psw, psb, lw, lb, wi2, wj2)


def _tc_combine2(h1, a2, s1, s2, mx, mn, cnt, psw, psb, lw, lb, gcw):
  def body(h_ref, a_ref, s1_ref, s2_ref, mx_ref, mn_ref, cnt_ref,
           psw_ref, psb_ref, lw_ref, lb_ref, gcw_ref, u_ref, dinv_ref):
    out2 = _pna_combine(h_ref[...], a_ref[...], s1_ref[...], s2_ref[...],
                        mx_ref[...], mn_ref[...], cnt_ref[...],
                        psw_ref[...], psb_ref[...], lw_ref[...], lb_ref[...])
    out2 = jax.nn.relu(out2)
    xw = _dot(out2, gcw_ref[...])
    dinv = lax.rsqrt(cnt_ref[...] + 1.0)
    dinv_ref[...] = dinv
    u_ref[...] = xw * dinv
  def bs(r, c):
    return pl.BlockSpec((r, c), lambda i: (i, 0))

  def ws(r, c):
    return pl.BlockSpec((r, c), lambda i: (0, 0))

  return pl.pallas_call(
      body,
      grid=(NPAD // BLKN,),
      in_specs=[bs(BLKN, 64), bs(BLKN, 64), bs(BLKN, 64), bs(BLKN, 64),
                bs(BLKN, 64), bs(BLKN, 64), bs(BLKN, 1),
                ws(384, 64), ws(1, 64), ws(64, 64), ws(1, 64), ws(64, 16)],
      out_specs=(bs(BLKN, 16), bs(BLKN, 1)),
      out_shape=(jax.ShapeDtypeStruct((NPAD, 16), jnp.float32),
                 jax.ShapeDtypeStruct((NPAD, 1), jnp.float32)),
  )(h1, a2, s1, s2, mx, mn, cnt, psw, psb, lw, lb, gcw)


def _tc_final(sg, u, dinv, xp, gcb, f1w, f1b, f2w, f2b):
  def body(sg_ref, u_ref, dinv_ref, x_ref, gcb_ref,
           f1w_ref, f1b_ref, f2w_ref, f2b_ref, o_ref):
    gout = dinv_ref[...] * (sg_ref[...] + u_ref[...]) + gcb_ref[...]
    h = jax.nn.relu(_dot(gout, f1w_ref[0:16]) + _dot(x_ref[...], f1w_ref[16:32])
                    + f1b_ref[...])
    o_ref[...] = _dot(h, f2w_ref[...]) + f2b_ref[...]
  def bs(r, c):
    return pl.BlockSpec((r, c), lambda i: (i, 0))

  def ws(r, c):
    return pl.BlockSpec((r, c), lambda i: (0, 0))

  return pl.pallas_call(
      body,
      grid=(NPAD // BLKN,),
      in_specs=[bs(BLKN, 16), bs(BLKN, 16), bs(BLKN, 1), bs(BLKN, 16),
                ws(1, 16), ws(32, 10), ws(1, 10), ws(10, 10), ws(1, 10)],
      out_specs=bs(BLKN, 10),
      out_shape=jax.ShapeDtypeStruct((NPAD, 10), jnp.float32),
  )(sg, u, dinv, xp, gcb, f1w, f1b, f2w, f2b)


def kernel(x, edge_index, edge_attr, p1ew, p1eb, p1prw, p1prb, p1psw, p1psb,
           p1lw, p1lb, p2ew, p2eb, p2prw, p2prb, p2psw, p2psb, p2lw, p2lb,
           gcw, gcb, f1w, f1b, f2w, f2b):
  src = edge_index[0]
  dst = edge_index[1]
  xp = jnp.pad(x, ((0, NPAD - N), (0, 0)))

  wi1, wj1, we1 = p1prw[0:16], p1prw[16:32], p1prw[32:48]
  wi2, wj2, we2 = p2prw[0:64], p2prw[64:128], p2prw[128:192]

  hist = _sc_hist(dst)
  p, off = _sc_offsets(hist)
  ea_t_flat = edge_attr.T.reshape(-1)
  src_perm, eac0, eac1, eac2, eac3 = _sc_scatter(src, dst, ea_t_flat, p, off)
  ea_perm = jnp.stack([eac0, eac1, eac2, eac3], axis=1)
  srcp2d = src_perm

  cnt = (p[1:NPAD + 1] - p[0:NPAD]).astype(jnp.float32)[:, None]

  a1, b1 = _tc_node_pre(xp, wi1, wj1)
  c1, c2 = _tc_edge_c(ea_perm, p1ew, p1eb[None, :], we1, p1prb[None, :],
                      p2ew, p2eb[None, :], we2, p2prb[None, :])

  s1, s2, mx, mn = _sc_accumulate(srcp2d, c1, b1, p, 16, 1024, True)
  h1, a2, b2 = _tc_combine1(xp, a1, s1, s2, mx, mn, cnt,
                            p1psw, p1psb[None, :], p1lw, p1lb[None, :],
                            wi2, wj2)

  s1b, s2b, mxb, mnb = _sc_accumulate(srcp2d, c2, b2, p, 64, 256, True)
  u, dinv = _tc_combine2(h1, a2, s1b, s2b, mxb, mnb, cnt,
                         p2psw, p2psb[None, :], p2lw, p2lb[None, :], gcw)

  sg = _sc_accumulate(srcp2d, None, u, p, 16, 1024, False)
  out = _tc_final(sg, u, dinv, xp, gcb[None, :], f1w, f1b[None, :],
                  f2w, f2b[None, :])
  return out[:N]
